# baseline (speedup 1.0000x reference)
import functools

import jax
import jax.numpy as jnp
from jax import lax
from jax.experimental import pallas as pl
from jax.experimental.pallas import tpu as pltpu
from jax.experimental.pallas import tpu_sc as plsc

_B = 8
_D = 21
_H = 512
_W = 512
_S = _H * _W
_NW = 32
_CHUNK = _B * _S // _NW
_K = 4096
_NSUB = _CHUNK // _K
_L = 16


def _sc_body(x_hbm, out_hbm, xbuf, ybuf, sem):
    cid = lax.axis_index("c")
    sid = lax.axis_index("s")
    wid = sid * 2 + cid
    b = wid // 4
    out_img = b * (_D * _S)
    in_base = wid * _CHUNK
    sp_base = in_base - b * _S

    iota = lax.broadcasted_iota(jnp.int32, (_L,), 0)
    ones_v = jnp.ones((_L,), jnp.float32)
    zeros_v = jnp.zeros((_L,), jnp.float32)

    def scatter_pass(val):
        def body(i, _):
            xv = xbuf[pl.ds(i * _L, _L)]
            idx = xv * _K + i * _L + iota
            plsc.store_scatter(ybuf, [idx], val)
            return 0
        lax.fori_loop(0, _K // _L, body, 0)

    def zero_body(i, _):
        ybuf[pl.ds(i * _L, _L)] = zeros_v
        return 0
    lax.fori_loop(0, _D * _K // _L, zero_body, 0)

    def sub_chunk(j, _):
        off = j * _K
        pltpu.sync_copy(x_hbm.at[pl.ds(in_base + off, _K)], xbuf)
        scatter_pass(ones_v)
        descs = [
            pltpu.async_copy(
                ybuf.at[pl.ds(ch * _K, _K)],
                out_hbm.at[pl.ds(out_img + ch * _S + sp_base + off, _K)],
                sem,
            )
            for ch in range(_D)
        ]
        for d in descs:
            d.wait()
        scatter_pass(zeros_v)
        return 0

    lax.fori_loop(0, _NSUB, sub_chunk, 0)


@jax.jit
def _sc_one_hot(x_flat):
    mesh = plsc.VectorSubcoreMesh(core_axis_name="c", subcore_axis_name="s")
    f = pl.kernel(
        _sc_body,
        out_type=jax.ShapeDtypeStruct((_B * _D * _S,), jnp.float32),
        mesh=mesh,
        scratch_types=[
            pltpu.VMEM((_K,), jnp.int32),
            pltpu.VMEM((_D * _K,), jnp.float32),
            pltpu.SemaphoreType.DMA,
        ],
        compiler_params=pltpu.CompilerParams(needs_layout_passes=False),
    )
    return f(x_flat)


def kernel(X_in, ones):
    del ones
    x_flat = X_in.reshape(-1).astype(jnp.int32)
    out = _sc_one_hot(x_flat)
    return out.reshape(_B, _D, _H, _W)
